# baseline (speedup 1.0000x reference)
import functools

import jax
import jax.numpy as jnp
from jax import lax
from jax.experimental import pallas as pl
from jax.experimental.pallas import tpu as pltpu
from jax.experimental.pallas import tpu_sc as plsc

_B = 16384
_D = 64
_NC = 2
_NS = 16
_NW = _NC * _NS
_BPW = _B // _NW
_L = 16
_DCHUNK = 16


def _mf_body(uid_hbm, mid_hbm, uemb_hbm, memb_hbm, ubias_hbm, mbias_hbm,
             out_hbm,
             uid_v, mid_v, ucols_v, mcols_v, ubias_v, mbias_v, out_v,
             sem_g, sem_b):
    wid = lax.axis_index("s") * _NC + lax.axis_index("c")
    base = wid * _BPW

    pltpu.sync_copy(uid_hbm.at[pl.ds(base, _BPW)], uid_v)
    pltpu.sync_copy(mid_hbm.at[pl.ds(base, _BPW)], mid_v)

    cbu = pltpu.async_copy(ubias_hbm.at[uid_v], ubias_v, sem_b)
    cbm = pltpu.async_copy(mbias_hbm.at[mid_v], mbias_v, sem_b)

    def fire(d0, carry):
        for dd in range(_DCHUNK):
            d = d0 * _DCHUNK + dd
            pltpu.async_copy(uemb_hbm.at[d].at[uid_v], ucols_v.at[d], sem_g)
            pltpu.async_copy(memb_hbm.at[d].at[mid_v], mcols_v.at[d], sem_g)
        return carry

    lax.fori_loop(0, _D // _DCHUNK, fire, 0)

    def drain(d0, carry):
        for dd in range(_DCHUNK):
            d = d0 * _DCHUNK + dd
            pltpu.make_async_copy(uemb_hbm.at[d].at[uid_v], ucols_v.at[d],
                                  sem_g).wait()
            pltpu.make_async_copy(memb_hbm.at[d].at[mid_v], mcols_v.at[d],
                                  sem_g).wait()
        return carry

    cbu.wait()
    cbm.wait()
    lax.fori_loop(0, _D // _DCHUNK, drain, 0)

    def group(g, carry):
        row0 = g * _L
        acc = ubias_v[pl.ds(row0, _L)] + mbias_v[pl.ds(row0, _L)]
        for d in range(_D):
            u = ucols_v[d, pl.ds(row0, _L)]
            m = mcols_v[d, pl.ds(row0, _L)]
            acc = acc + u * m
        out_v[pl.ds(row0, _L)] = acc
        return carry

    lax.fori_loop(0, _BPW // _L, group, 0)
    pltpu.sync_copy(out_v, out_hbm.at[pl.ds(base, _BPW)])


@functools.partial(jax.jit, donate_argnums=())
def kernel(user_ids, movie_ids, user_emb, movie_emb, user_bias, movie_bias):
    run = pl.kernel(
        _mf_body,
        out_type=jax.ShapeDtypeStruct((_B,), jnp.float32),
        mesh=plsc.VectorSubcoreMesh(core_axis_name="c", subcore_axis_name="s"),
        compiler_params=pltpu.CompilerParams(
            needs_layout_passes=False, use_tc_tiling_on_sc=False),
        scratch_types=[
            pltpu.VMEM((_BPW,), jnp.int32),
            pltpu.VMEM((_BPW,), jnp.int32),
            pltpu.VMEM((_D, _BPW), jnp.float32),
            pltpu.VMEM((_D, _BPW), jnp.float32),
            pltpu.VMEM((_BPW,), jnp.float32),
            pltpu.VMEM((_BPW,), jnp.float32),
            pltpu.VMEM((_BPW,), jnp.float32),
            pltpu.SemaphoreType.DMA,
            pltpu.SemaphoreType.DMA,
        ],
    )
    return run(user_ids.astype(jnp.int32), movie_ids.astype(jnp.int32),
               user_emb.T, movie_emb.T,
               user_bias.reshape(-1), movie_bias.reshape(-1))
